# baseline (speedup 1.0000x reference)
import functools

import jax
import jax.numpy as jnp
import numpy as np
from jax import lax
from jax.experimental import pallas as pl
from jax.experimental.pallas import tpu as pltpu
from jax.experimental.pallas import tpu_sc as plsc

N_NODES = 10000
N_EDGES = 160000
HIDDEN = 256
N_MOLS = 200

NC = 2
NS = 16
OWN = 5120
NPAD = NC * OWN
EPT = N_EDGES // NS
CHUNK = 128
NFULL = EPT // CHUNK
TAIL = EPT - NFULL * CHUNK
OWN_T = NPAD // (NC * NS)
ACC_T = OWN_T + 16
EBLK = 1280

M_ACC = 384
M_TRASH = 200

_MESH = plsc.VectorSubcoreMesh(core_axis_name="c", subcore_axis_name="s")

_f32 = jnp.float32
_i32 = jnp.int32


def _sc_deg_body(dst_hbm, out_hbm, dbuf, sbuf, ones_v, z1, acc, sem):
    c = lax.axis_index("c")
    t = lax.axis_index("s")
    lo = c * OWN
    hi = OWN + c * (N_NODES - OWN)
    iota16 = lax.iota(_i32, 16)

    one16 = jnp.ones((16,), _f32)
    zero16 = jnp.zeros((16,), _f32)
    for j in range(CHUNK // 16):
        ones_v[pl.ds(16 * j, 16)] = one16
    for j in range(336 // 16):
        z1[pl.ds(16 * j, 16)] = zero16
    pltpu.sync_copy(z1, acc.at[pl.ds(t * 336, 336)])
    plsc.subcore_barrier()

    def flush(f, carry):
        eb = t * EPT + f * CHUNK
        pltpu.sync_copy(dst_hbm.at[pl.ds(eb, CHUNK)], dbuf)
        for j in range(CHUNK // 16):
            dv = dbuf[pl.ds(16 * j, 16)]
            owned = (dv >= lo) & (dv < hi)
            sbuf[pl.ds(16 * j, 16)] = jnp.where(owned, dv - lo,
                                                OWN + 16 * j + iota16)
        pltpu.sync_copy(ones_v, acc.at[sbuf], add=True)
        return carry

    lax.fori_loop(0, NFULL, flush, 0)

    eb = t * EPT + NFULL * CHUNK
    pltpu.sync_copy(dst_hbm.at[pl.ds(eb, TAIL)], dbuf.at[pl.ds(0, TAIL)])
    dv = dbuf[pl.ds(0, 16)]
    owned = (dv >= lo) & (dv < hi)
    sbuf[pl.ds(0, 16)] = jnp.where(owned, dv - lo, OWN + iota16)
    for j in range(1, CHUNK // 16):
        sbuf[pl.ds(16 * j, 16)] = OWN + 16 * j + iota16
    pltpu.sync_copy(ones_v, acc.at[sbuf], add=True)

    plsc.subcore_barrier()

    @pl.when(t < 10)
    def _():
        pltpu.sync_copy(acc.at[pl.ds(t * 512, 512)],
                        out_hbm.at[pl.ds(c * OWN + t * 512, 512)])


_sc_deg = functools.partial(
    pl.kernel,
    out_type=jax.ShapeDtypeStruct((NPAD,), _f32),
    mesh=_MESH,
    scratch_types=[
        pltpu.VMEM((CHUNK,), _i32),
        pltpu.VMEM((CHUNK,), _i32),
        pltpu.VMEM((CHUNK,), _f32),
        pltpu.VMEM((336,), _f32),
        pltpu.VMEM_SHARED((NS * 336,), _f32),
        pltpu.SemaphoreType.DMA,
    ],
)(_sc_deg_body)


def _sc_agg_body(g_hbm, src_hbm, dst_hbm, zer_hbm, out_hbm,
                 srcbuf, dstbuf, lst, gbuf, abuf, pkblk, csblk, fgblk,
                 staging, acc, sem):
    c = lax.axis_index("c")
    t = lax.axis_index("s")
    w = c * NS + t
    lo = w * OWN_T
    iota16 = lax.iota(_i32, 16)
    one16 = iota16 * 0 + 1
    _gdn = lax.GatherDimensionNumbers(offset_dims=(), collapsed_slice_dims=(0,),
                                      start_index_map=(0,))

    def _perm(v, idx):
        return lax.gather(v, idx.reshape(16, 1), _gdn, (1,),
                          mode=lax.GatherScatterMode.PROMISE_IN_BOUNDS)

    sh = [jnp.maximum(iota16 - k, 0) for k in (1, 2, 4, 8)]
    gm = [one16 - lax.shift_right_logical(iota16 - k, 31) for k in (1, 2, 4, 8)]
    NG = EBLK // 16

    pltpu.sync_copy(zer_hbm, acc)

    def rmw_flush(f):
        for g in range(CHUNK // 16):
            v2 = lst[pl.ds(f * CHUNK + 16 * g, 16)]
            gbuf[pl.ds(16 * g, 16)] = lax.bitwise_and(v2, 16383)
            abuf[pl.ds(16 * g, 16)] = lax.bitwise_and(lax.shift_right_logical(v2, 14), 1023)
        pltpu.async_copy(g_hbm.at[gbuf], staging, sem).wait()

        def grp(g, carry):
            dv16 = abuf[pl.ds(g * 16, 16)]
            for k in range(16):
                dloc2 = dv16[k]
                e = g * 16 + k
                for j in range(HIDDEN // 16):
                    acc[dloc2, pl.ds(16 * j, 16)] = (
                        acc[dloc2, pl.ds(16 * j, 16)]
                        + staging[e, pl.ds(16 * j, 16)])
            return carry

        lax.fori_loop(0, CHUNK // 16, grp, 0)

    def blk(b, cnt):
        pltpu.sync_copy(src_hbm.at[pl.ds(b * EBLK, EBLK)], srcbuf)
        pltpu.sync_copy(dst_hbm.at[pl.ds(b * EBLK, EBLK)], dstbuf)

        def la(i, carry):
            dv = dstbuf[pl.ds(i * 16, 16)]
            sv = srcbuf[pl.ds(i * 16, 16)]
            dvl = dv - lo
            m = one16 - lax.shift_right_logical(
                lax.bitwise_or(dvl, (OWN_T - 1) - dvl), 31)
            mf = iota16 * 0 - m
            dloc = lax.bitwise_or(
                lax.bitwise_and(dvl, mf),
                lax.bitwise_and(OWN_T + iota16, lax.bitwise_xor(mf, -1)))
            pkblk[pl.ds(i * 16, 16)] = (lax.shift_left(m, 24)
                                        + lax.shift_left(dloc, 14) + sv)
            return carry

        lax.fori_loop(0, NG, la, 0)

        def lb(i, cnt):
            cs = lax.shift_right_logical(pkblk[pl.ds(i * 16, 16)], 24)
            for si, g01 in zip(sh, gm):
                cs = cs + _perm(cs, si) * g01
            csblk[pl.ds(i * 16, 16)] = cs
            return cnt + cs[15]

        cnt0 = cnt
        cnt = lax.fori_loop(0, NG, lb, cnt)

        def lc(i, carry):
            cs2 = csblk[pl.ds(i * 16, 16)]

            @pl.when(cs2[15] > 0)
            def _():
                fg = iota16 * 0
                for step in (8, 4):
                    cand = fg + step
                    okm = one16 - lax.shift_right_logical(
                        iota16 - _perm(cs2, cand - one16), 31)
                    fg = fg + step * okm
                fgblk[pl.ds(i * 16, 16)] = fg

            return carry

        lax.fori_loop(0, NG, lc, 0)

        def ld(i, cnt2):
            cs3 = csblk[pl.ds(i * 16, 16)]

            @pl.when(cs3[15] > 0)
            def _():
                fg2 = fgblk[pl.ds(i * 16, 16)]
                for step in (2, 1):
                    cand = fg2 + step
                    okm = one16 - lax.shift_right_logical(
                        iota16 - _perm(cs3, cand - one16), 31)
                    fg2 = fg2 + step * okm
                lst[pl.ds(cnt2, 16)] = _perm(pkblk[pl.ds(i * 16, 16)], fg2)

            return cnt2 + cs3[15]

        lax.fori_loop(0, NG, ld, cnt0)

        def fl(f2, cnt):
            @pl.when(cnt >= CHUNK)
            def _():
                rmw_flush(0)
                for gmv in range((EBLK + CHUNK) // 16):
                    v3 = lst[pl.ds(CHUNK + 16 * gmv, 16)]
                    lst[pl.ds(16 * gmv, 16)] = v3

            return jnp.where(cnt >= CHUNK, cnt - CHUNK, cnt)

        return lax.fori_loop(0, EBLK // CHUNK + 1, fl, cnt)

    cnt = lax.fori_loop(0, N_EDGES // EBLK, blk, jnp.zeros((), _i32))

    junk = lax.shift_left(OWN_T + iota16, 14) + iota16
    for k2 in range(CHUNK // 16):
        lst[pl.ds(cnt + 16 * k2, 16)] = junk + 16 * k2
    rmw_flush(0)

    pltpu.sync_copy(acc.at[pl.ds(0, OWN_T)],
                    out_hbm.at[pl.ds(w * OWN_T, OWN_T)])


_sc_agg = functools.partial(
    pl.kernel,
    out_type=jax.ShapeDtypeStruct((NPAD, HIDDEN), _f32),
    mesh=_MESH,
    scratch_types=[
        pltpu.VMEM((EBLK,), _i32),
        pltpu.VMEM((EBLK,), _i32),
        pltpu.VMEM((EBLK + 2 * CHUNK, ), _i32),
        pltpu.VMEM((CHUNK,), _i32),
        pltpu.VMEM((CHUNK,), _i32),
        pltpu.VMEM((EBLK,), _i32),
        pltpu.VMEM((EBLK,), _i32),
        pltpu.VMEM((EBLK,), _i32),
        pltpu.VMEM((CHUNK, HIDDEN), _f32),
        pltpu.VMEM((ACC_T, HIDDEN), _f32),
        pltpu.SemaphoreType.DMA,
    ],
)(_sc_agg_body)


def _sc_mid_body(a_hbm, seg_hbm, out_hbm, staging, segbuf, acc, sem):
    c = lax.axis_index("c")
    t = lax.axis_index("s")
    w = c * NS + t
    base = w * 320
    z16 = jnp.zeros((16,), _f32)

    def zrow(i, carry):
        for j in range(HIDDEN // 16):
            acc[i, pl.ds(16 * j, 16)] = z16
        return carry

    lax.fori_loop(0, N_MOLS, zrow, 0)

    def do_chunk(off, sz):
        pltpu.sync_copy(seg_hbm.at[pl.ds(base + off, sz)], segbuf.at[pl.ds(0, sz)])
        pltpu.sync_copy(a_hbm.at[pl.ds(base + off, sz)], staging.at[pl.ds(0, sz)])

        def grp(g, carry):
            sg16 = segbuf[pl.ds(g * 16, 16)]
            for k in range(16):
                dloc = sg16[k]
                e = g * 16 + k
                for j in range(HIDDEN // 16):
                    acc[dloc, pl.ds(16 * j, 16)] = (
                        acc[dloc, pl.ds(16 * j, 16)]
                        + staging[e, pl.ds(16 * j, 16)])
            return carry

        lax.fori_loop(0, sz // 16, grp, 0)

    @pl.when(w < 31)
    def _():
        do_chunk(0, 128)
        do_chunk(128, 128)
        do_chunk(256, 64)

    @pl.when(w == 31)
    def _():
        do_chunk(0, 80)

    pltpu.sync_copy(acc, out_hbm.at[w])


_sc_mid = functools.partial(
    pl.kernel,
    out_type=jax.ShapeDtypeStruct((NC * NS, N_MOLS, HIDDEN), _f32),
    mesh=_MESH,
    scratch_types=[
        pltpu.VMEM((CHUNK, HIDDEN), _f32),
        pltpu.VMEM((CHUNK,), _i32),
        pltpu.VMEM((N_MOLS, HIDDEN), _f32),
        pltpu.SemaphoreType.DMA,
    ],
)(_sc_mid_body)


ROWS_BLK = 2048
GRID = NPAD // ROWS_BLK


def _lorentz_rescale(y, es):
    time = jax.nn.sigmoid(y[:, :1]) * es + 1.1
    xn = y[:, 1:]
    scale = (time * time - 1.0) / jnp.clip(
        jnp.sum(xn * xn, axis=-1, keepdims=True), 1e-8, None)
    return jnp.concatenate([time, xn * jnp.sqrt(scale)], axis=-1)


def _tc_first_body(x_ref, cnt_ref, w_ref, b_ref, s_ref, o_ref):
    x = x_ref[...]
    nrm = jnp.sqrt(jnp.clip(jnp.sum(x * x, axis=-1, keepdims=True), 1e-8, None))
    ep = jnp.exp(nrm)
    en = jnp.exp(-nrm)
    h = jnp.concatenate([0.5 * (ep + en), (0.5 * (ep - en)) / nrm * x], axis=-1)
    y = lax.dot_general(h, w_ref[...], (((1,), (0,)), ((), ())),
                        preferred_element_type=_f32) + b_ref[...]
    h = _lorentz_rescale(y, jnp.exp(s_ref[0, 0]))
    o_ref[...] = h * lax.rsqrt(cnt_ref[...] + 1.0)


def _tc_mid_body(S_ref, g_ref, cnt_ref, w_ref, b_ref, s_ref, o_ref):
    v = S_ref[...] + g_ref[...]
    v0 = v[:, :1]
    inner = jnp.sum(v * v, axis=-1, keepdims=True) - 2.0 * v0 * v0
    a = v * lax.rsqrt(jnp.clip(jnp.abs(inner), 1e-8, None))
    a = jnp.maximum(a, 0.0)
    y = lax.dot_general(a, w_ref[...], (((1,), (0,)), ((), ())),
                        preferred_element_type=_f32) + b_ref[...]
    h = _lorentz_rescale(y, jnp.exp(s_ref[0, 0]))
    o_ref[...] = h * lax.rsqrt(cnt_ref[...] + 1.0)


def _tc_norm_body(S_ref, g_ref, o_ref):
    v = S_ref[...] + g_ref[...]
    v0 = v[:, :1]
    inner = jnp.sum(v * v, axis=-1, keepdims=True) - 2.0 * v0 * v0
    o_ref[...] = v * lax.rsqrt(jnp.clip(jnp.abs(inner), 1e-8, None))


def _tc_final_body(p_ref, o_ref):
    v = p_ref[0]
    for k in range(1, NC * NS):
        v = v + p_ref[k]
    v0 = v[:, :1]
    inner = jnp.sum(v * v, axis=-1, keepdims=True) - 2.0 * v0 * v0
    o_ref[...] = v * lax.rsqrt(jnp.clip(jnp.abs(inner), 1e-8, None))


def _rows_spec(width):
    return pl.BlockSpec((ROWS_BLK, width), lambda i: (i, 0))


def _full_spec(shape):
    return pl.BlockSpec(shape, lambda i: tuple(0 for _ in shape))


def _tc_first(x, cnt2d, W, b2d, s2d):
    return pl.pallas_call(
        _tc_first_body,
        grid=(GRID,),
        in_specs=[_rows_spec(x.shape[1]), _rows_spec(1),
                  _full_spec(W.shape), _full_spec(b2d.shape), _full_spec(s2d.shape)],
        out_specs=_rows_spec(HIDDEN),
        out_shape=jax.ShapeDtypeStruct((NPAD, HIDDEN), _f32),
    )(x, cnt2d, W, b2d, s2d)


def _tc_mid(S, g, cnt2d, W, b2d, s2d):
    return pl.pallas_call(
        _tc_mid_body,
        grid=(GRID,),
        in_specs=[_rows_spec(HIDDEN), _rows_spec(HIDDEN), _rows_spec(1),
                  _full_spec(W.shape), _full_spec(b2d.shape), _full_spec(s2d.shape)],
        out_specs=_rows_spec(HIDDEN),
        out_shape=jax.ShapeDtypeStruct((NPAD, HIDDEN), _f32),
    )(S, g, cnt2d, W, b2d, s2d)


def _tc_norm(S, g):
    return pl.pallas_call(
        _tc_norm_body,
        grid=(GRID,),
        in_specs=[_rows_spec(HIDDEN), _rows_spec(HIDDEN)],
        out_specs=_rows_spec(HIDDEN),
        out_shape=jax.ShapeDtypeStruct((NPAD, HIDDEN), _f32),
    )(S, g)


def _tc_final(p):
    return pl.pallas_call(
        _tc_final_body,
        in_specs=[pl.BlockSpec((NC * NS, N_MOLS, HIDDEN), lambda: (0, 0, 0))],
        out_specs=pl.BlockSpec((N_MOLS, HIDDEN), lambda: (0, 0)),
        out_shape=jax.ShapeDtypeStruct((N_MOLS, HIDDEN), _f32),
    )(p)


def kernel(x, edge_index, segment_ids, W0, b0, s0, W1, b1, s1, W2, b2, s2,
           W3, b3, s3):
    src = edge_index[0]
    dst = edge_index[1]
    counts = _sc_deg(dst)
    cnt2d = counts.reshape(NPAD, 1)
    xp = jnp.pad(x, ((0, NPAD - N_NODES), (0, 0)))

    def prep(b, s):
        return b.reshape(1, HIDDEN), s.reshape(1, 1)

    zer = jnp.zeros((ACC_T, HIDDEN), _f32)
    b0r, s0r = prep(b0, s0)
    g = _tc_first(xp, cnt2d, W0, b0r, s0r)
    for W, b, s in ((W1, b1, s1), (W2, b2, s2), (W3, b3, s3)):
        S = _sc_agg(g, src, dst, zer)
        br, sr = prep(b, s)
        g = _tc_mid(S, g, cnt2d, W, br, sr)
    S = _sc_agg(g, src, dst, zer)
    a = _tc_norm(S, g)
    partials = _sc_mid(a, segment_ids)
    return _tc_final(partials)


# speedup vs baseline: 1.2658x; 1.2658x over previous
import functools

import jax
import jax.numpy as jnp
import numpy as np
from jax import lax
from jax.experimental import pallas as pl
from jax.experimental.pallas import tpu as pltpu
from jax.experimental.pallas import tpu_sc as plsc

N_NODES = 10000
N_EDGES = 160000
HIDDEN = 256
N_MOLS = 200

NC = 2
NS = 16
OWN = 5120
NPAD = NC * OWN
EPT = N_EDGES // NS
CHUNK = 128
NFULL = EPT // CHUNK
TAIL = EPT - NFULL * CHUNK
OWN_T = NPAD // (NC * NS)
ACC_T = OWN_T + 16
EBLK = 1280

M_ACC = 384
M_TRASH = 200

_MESH = plsc.VectorSubcoreMesh(core_axis_name="c", subcore_axis_name="s")

_f32 = jnp.float32
_i32 = jnp.int32


def _sc_deg_body(dst_hbm, out_hbm, dbuf, sbuf, ones_v, z1, acc, sem):
    c = lax.axis_index("c")
    t = lax.axis_index("s")
    lo = c * OWN
    hi = OWN + c * (N_NODES - OWN)
    iota16 = lax.iota(_i32, 16)

    one16 = jnp.ones((16,), _f32)
    zero16 = jnp.zeros((16,), _f32)
    for j in range(CHUNK // 16):
        ones_v[pl.ds(16 * j, 16)] = one16
    for j in range(336 // 16):
        z1[pl.ds(16 * j, 16)] = zero16
    pltpu.sync_copy(z1, acc.at[pl.ds(t * 336, 336)])
    plsc.subcore_barrier()

    def flush(f, carry):
        eb = t * EPT + f * CHUNK
        pltpu.sync_copy(dst_hbm.at[pl.ds(eb, CHUNK)], dbuf)
        for j in range(CHUNK // 16):
            dv = dbuf[pl.ds(16 * j, 16)]
            owned = (dv >= lo) & (dv < hi)
            sbuf[pl.ds(16 * j, 16)] = jnp.where(owned, dv - lo,
                                                OWN + 16 * j + iota16)
        pltpu.sync_copy(ones_v, acc.at[sbuf], add=True)
        return carry

    lax.fori_loop(0, NFULL, flush, 0)

    eb = t * EPT + NFULL * CHUNK
    pltpu.sync_copy(dst_hbm.at[pl.ds(eb, TAIL)], dbuf.at[pl.ds(0, TAIL)])
    dv = dbuf[pl.ds(0, 16)]
    owned = (dv >= lo) & (dv < hi)
    sbuf[pl.ds(0, 16)] = jnp.where(owned, dv - lo, OWN + iota16)
    for j in range(1, CHUNK // 16):
        sbuf[pl.ds(16 * j, 16)] = OWN + 16 * j + iota16
    pltpu.sync_copy(ones_v, acc.at[sbuf], add=True)

    plsc.subcore_barrier()

    @pl.when(t < 10)
    def _():
        pltpu.sync_copy(acc.at[pl.ds(t * 512, 512)],
                        out_hbm.at[pl.ds(c * OWN + t * 512, 512)])


_sc_deg = functools.partial(
    pl.kernel,
    out_type=jax.ShapeDtypeStruct((NPAD,), _f32),
    mesh=_MESH,
    scratch_types=[
        pltpu.VMEM((CHUNK,), _i32),
        pltpu.VMEM((CHUNK,), _i32),
        pltpu.VMEM((CHUNK,), _f32),
        pltpu.VMEM((336,), _f32),
        pltpu.VMEM_SHARED((NS * 336,), _f32),
        pltpu.SemaphoreType.DMA,
    ],
)(_sc_deg_body)


def _sc_agg_body(g_hbm, src_hbm, dst_hbm, zer_hbm, out_hbm,
                 srcbuf, dstbuf, lst, gbuf, abuf, pkblk, csblk, fgblk,
                 staging, acc, sem):
    c = lax.axis_index("c")
    t = lax.axis_index("s")
    w = c * NS + t
    lo = w * OWN_T
    iota16 = lax.iota(_i32, 16)
    one16 = iota16 * 0 + 1
    _gdn = lax.GatherDimensionNumbers(offset_dims=(), collapsed_slice_dims=(0,),
                                      start_index_map=(0,))

    def _perm(v, idx):
        return lax.gather(v, idx.reshape(16, 1), _gdn, (1,),
                          mode=lax.GatherScatterMode.PROMISE_IN_BOUNDS)

    sh = [jnp.maximum(iota16 - k, 0) for k in (1, 2, 4, 8)]
    gm = [one16 - lax.shift_right_logical(iota16 - k, 31) for k in (1, 2, 4, 8)]
    NG = EBLK // 16

    pltpu.sync_copy(zer_hbm, acc)

    def rmw_flush(f):
        for g in range(CHUNK // 16):
            v2 = lst[pl.ds(f * CHUNK + 16 * g, 16)]
            gbuf[pl.ds(16 * g, 16)] = lax.bitwise_and(v2, 16383)
            abuf[pl.ds(16 * g, 16)] = lax.bitwise_and(lax.shift_right_logical(v2, 14), 1023)
        pltpu.async_copy(g_hbm.at[gbuf], staging, sem).wait()

        def grp(g, carry):
            dv16 = abuf[pl.ds(g * 16, 16)]
            for k in range(16):
                dloc2 = dv16[k]
                e = g * 16 + k
                for j in range(HIDDEN // 16):
                    acc[dloc2, pl.ds(16 * j, 16)] = (
                        acc[dloc2, pl.ds(16 * j, 16)]
                        + staging[e, pl.ds(16 * j, 16)])
            return carry

        lax.fori_loop(0, CHUNK // 16, grp, 0)

    def blk(b, cnt):
        pltpu.sync_copy(src_hbm.at[pl.ds(b * EBLK, EBLK)], srcbuf)
        pltpu.sync_copy(dst_hbm.at[pl.ds(b * EBLK, EBLK)], dstbuf)

        def la(i, carry):
            dv = dstbuf[pl.ds(i * 16, 16)]
            sv = srcbuf[pl.ds(i * 16, 16)]
            dvl = dv - lo
            m = one16 - lax.shift_right_logical(
                lax.bitwise_or(dvl, (OWN_T - 1) - dvl), 31)
            mf = iota16 * 0 - m
            dloc = lax.bitwise_or(
                lax.bitwise_and(dvl, mf),
                lax.bitwise_and(OWN_T + iota16, lax.bitwise_xor(mf, -1)))
            pkblk[pl.ds(i * 16, 16)] = (lax.shift_left(m, 24)
                                        + lax.shift_left(dloc, 14) + sv)
            return carry

        lax.fori_loop(0, NG, la, 0)

        def lb(i, cnt):
            cs = lax.shift_right_logical(pkblk[pl.ds(i * 16, 16)], 24)
            for si, g01 in zip(sh, gm):
                cs = cs + _perm(cs, si) * g01
            csblk[pl.ds(i * 16, 16)] = cs
            return cnt + cs[15]

        cnt0 = cnt
        cnt = lax.fori_loop(0, NG, lb, cnt)

        def ld(i, cnt2):
            cs3 = csblk[pl.ds(i * 16, 16)]
            fg2 = iota16 * 0
            for step in (8, 4, 2, 1):
                cand = fg2 + step
                okm = one16 - lax.shift_right_logical(
                    iota16 - _perm(cs3, cand - one16), 31)
                fg2 = fg2 + step * okm
            lst[pl.ds(cnt2, 16)] = _perm(pkblk[pl.ds(i * 16, 16)], fg2)
            return cnt2 + cs3[15]

        lax.fori_loop(0, NG, ld, cnt0)

        def fl(f2, cnt):
            @pl.when(cnt >= CHUNK)
            def _():
                rmw_flush(0)
                for gmv in range((EBLK + CHUNK) // 16):
                    v3 = lst[pl.ds(CHUNK + 16 * gmv, 16)]
                    lst[pl.ds(16 * gmv, 16)] = v3

            return jnp.where(cnt >= CHUNK, cnt - CHUNK, cnt)

        return lax.fori_loop(0, EBLK // CHUNK + 1, fl, cnt)

    cnt = lax.fori_loop(0, N_EDGES // EBLK, blk, jnp.zeros((), _i32))

    junk = lax.shift_left(OWN_T + iota16, 14) + iota16
    for k2 in range(CHUNK // 16):
        lst[pl.ds(cnt + 16 * k2, 16)] = junk + 16 * k2
    rmw_flush(0)

    pltpu.sync_copy(acc.at[pl.ds(0, OWN_T)],
                    out_hbm.at[pl.ds(w * OWN_T, OWN_T)])


_sc_agg = functools.partial(
    pl.kernel,
    out_type=jax.ShapeDtypeStruct((NPAD, HIDDEN), _f32),
    mesh=_MESH,
    scratch_types=[
        pltpu.VMEM((EBLK,), _i32),
        pltpu.VMEM((EBLK,), _i32),
        pltpu.VMEM((EBLK + 2 * CHUNK, ), _i32),
        pltpu.VMEM((CHUNK,), _i32),
        pltpu.VMEM((CHUNK,), _i32),
        pltpu.VMEM((EBLK,), _i32),
        pltpu.VMEM((EBLK,), _i32),
        pltpu.VMEM((EBLK,), _i32),
        pltpu.VMEM((CHUNK, HIDDEN), _f32),
        pltpu.VMEM((ACC_T, HIDDEN), _f32),
        pltpu.SemaphoreType.DMA,
    ],
)(_sc_agg_body)


def _sc_mid_body(a_hbm, seg_hbm, out_hbm, staging, segbuf, acc, sem):
    c = lax.axis_index("c")
    t = lax.axis_index("s")
    w = c * NS + t
    base = w * 320
    z16 = jnp.zeros((16,), _f32)

    def zrow(i, carry):
        for j in range(HIDDEN // 16):
            acc[i, pl.ds(16 * j, 16)] = z16
        return carry

    lax.fori_loop(0, N_MOLS, zrow, 0)

    def do_chunk(off, sz):
        pltpu.sync_copy(seg_hbm.at[pl.ds(base + off, sz)], segbuf.at[pl.ds(0, sz)])
        pltpu.sync_copy(a_hbm.at[pl.ds(base + off, sz)], staging.at[pl.ds(0, sz)])

        def grp(g, carry):
            sg16 = segbuf[pl.ds(g * 16, 16)]
            for k in range(16):
                dloc = sg16[k]
                e = g * 16 + k
                for j in range(HIDDEN // 16):
                    acc[dloc, pl.ds(16 * j, 16)] = (
                        acc[dloc, pl.ds(16 * j, 16)]
                        + staging[e, pl.ds(16 * j, 16)])
            return carry

        lax.fori_loop(0, sz // 16, grp, 0)

    @pl.when(w < 31)
    def _():
        do_chunk(0, 128)
        do_chunk(128, 128)
        do_chunk(256, 64)

    @pl.when(w == 31)
    def _():
        do_chunk(0, 80)

    pltpu.sync_copy(acc, out_hbm.at[w])


_sc_mid = functools.partial(
    pl.kernel,
    out_type=jax.ShapeDtypeStruct((NC * NS, N_MOLS, HIDDEN), _f32),
    mesh=_MESH,
    scratch_types=[
        pltpu.VMEM((CHUNK, HIDDEN), _f32),
        pltpu.VMEM((CHUNK,), _i32),
        pltpu.VMEM((N_MOLS, HIDDEN), _f32),
        pltpu.SemaphoreType.DMA,
    ],
)(_sc_mid_body)


ROWS_BLK = 2048
GRID = NPAD // ROWS_BLK


def _lorentz_rescale(y, es):
    time = jax.nn.sigmoid(y[:, :1]) * es + 1.1
    xn = y[:, 1:]
    scale = (time * time - 1.0) / jnp.clip(
        jnp.sum(xn * xn, axis=-1, keepdims=True), 1e-8, None)
    return jnp.concatenate([time, xn * jnp.sqrt(scale)], axis=-1)


def _tc_first_body(x_ref, cnt_ref, w_ref, b_ref, s_ref, o_ref):
    x = x_ref[...]
    nrm = jnp.sqrt(jnp.clip(jnp.sum(x * x, axis=-1, keepdims=True), 1e-8, None))
    ep = jnp.exp(nrm)
    en = jnp.exp(-nrm)
    h = jnp.concatenate([0.5 * (ep + en), (0.5 * (ep - en)) / nrm * x], axis=-1)
    y = lax.dot_general(h, w_ref[...], (((1,), (0,)), ((), ())),
                        preferred_element_type=_f32) + b_ref[...]
    h = _lorentz_rescale(y, jnp.exp(s_ref[0, 0]))
    o_ref[...] = h * lax.rsqrt(cnt_ref[...] + 1.0)


def _tc_mid_body(S_ref, g_ref, cnt_ref, w_ref, b_ref, s_ref, o_ref):
    v = S_ref[...] + g_ref[...]
    v0 = v[:, :1]
    inner = jnp.sum(v * v, axis=-1, keepdims=True) - 2.0 * v0 * v0
    a = v * lax.rsqrt(jnp.clip(jnp.abs(inner), 1e-8, None))
    a = jnp.maximum(a, 0.0)
    y = lax.dot_general(a, w_ref[...], (((1,), (0,)), ((), ())),
                        preferred_element_type=_f32) + b_ref[...]
    h = _lorentz_rescale(y, jnp.exp(s_ref[0, 0]))
    o_ref[...] = h * lax.rsqrt(cnt_ref[...] + 1.0)


def _tc_norm_body(S_ref, g_ref, o_ref):
    v = S_ref[...] + g_ref[...]
    v0 = v[:, :1]
    inner = jnp.sum(v * v, axis=-1, keepdims=True) - 2.0 * v0 * v0
    o_ref[...] = v * lax.rsqrt(jnp.clip(jnp.abs(inner), 1e-8, None))


def _tc_final_body(p_ref, o_ref):
    v = p_ref[0]
    for k in range(1, NC * NS):
        v = v + p_ref[k]
    v0 = v[:, :1]
    inner = jnp.sum(v * v, axis=-1, keepdims=True) - 2.0 * v0 * v0
    o_ref[...] = v * lax.rsqrt(jnp.clip(jnp.abs(inner), 1e-8, None))


def _rows_spec(width):
    return pl.BlockSpec((ROWS_BLK, width), lambda i: (i, 0))


def _full_spec(shape):
    return pl.BlockSpec(shape, lambda i: tuple(0 for _ in shape))


def _tc_first(x, cnt2d, W, b2d, s2d):
    return pl.pallas_call(
        _tc_first_body,
        grid=(GRID,),
        in_specs=[_rows_spec(x.shape[1]), _rows_spec(1),
                  _full_spec(W.shape), _full_spec(b2d.shape), _full_spec(s2d.shape)],
        out_specs=_rows_spec(HIDDEN),
        out_shape=jax.ShapeDtypeStruct((NPAD, HIDDEN), _f32),
    )(x, cnt2d, W, b2d, s2d)


def _tc_mid(S, g, cnt2d, W, b2d, s2d):
    return pl.pallas_call(
        _tc_mid_body,
        grid=(GRID,),
        in_specs=[_rows_spec(HIDDEN), _rows_spec(HIDDEN), _rows_spec(1),
                  _full_spec(W.shape), _full_spec(b2d.shape), _full_spec(s2d.shape)],
        out_specs=_rows_spec(HIDDEN),
        out_shape=jax.ShapeDtypeStruct((NPAD, HIDDEN), _f32),
    )(S, g, cnt2d, W, b2d, s2d)


def _tc_norm(S, g):
    return pl.pallas_call(
        _tc_norm_body,
        grid=(GRID,),
        in_specs=[_rows_spec(HIDDEN), _rows_spec(HIDDEN)],
        out_specs=_rows_spec(HIDDEN),
        out_shape=jax.ShapeDtypeStruct((NPAD, HIDDEN), _f32),
    )(S, g)


def _tc_final(p):
    return pl.pallas_call(
        _tc_final_body,
        in_specs=[pl.BlockSpec((NC * NS, N_MOLS, HIDDEN), lambda: (0, 0, 0))],
        out_specs=pl.BlockSpec((N_MOLS, HIDDEN), lambda: (0, 0)),
        out_shape=jax.ShapeDtypeStruct((N_MOLS, HIDDEN), _f32),
    )(p)


def kernel(x, edge_index, segment_ids, W0, b0, s0, W1, b1, s1, W2, b2, s2,
           W3, b3, s3):
    src = edge_index[0]
    dst = edge_index[1]
    counts = _sc_deg(dst)
    cnt2d = counts.reshape(NPAD, 1)
    xp = jnp.pad(x, ((0, NPAD - N_NODES), (0, 0)))

    def prep(b, s):
        return b.reshape(1, HIDDEN), s.reshape(1, 1)

    zer = jnp.zeros((ACC_T, HIDDEN), _f32)
    b0r, s0r = prep(b0, s0)
    g = _tc_first(xp, cnt2d, W0, b0r, s0r)
    for W, b, s in ((W1, b1, s1), (W2, b2, s2), (W3, b3, s3)):
        S = _sc_agg(g, src, dst, zer)
        br, sr = prep(b, s)
        g = _tc_mid(S, g, cnt2d, W, br, sr)
    S = _sc_agg(g, src, dst, zer)
    a = _tc_norm(S, g)
    partials = _sc_mid(a, segment_ids)
    return _tc_final(partials)


# speedup vs baseline: 1.3200x; 1.0428x over previous
import functools

import jax
import jax.numpy as jnp
import numpy as np
from jax import lax
from jax.experimental import pallas as pl
from jax.experimental.pallas import tpu as pltpu
from jax.experimental.pallas import tpu_sc as plsc

N_NODES = 10000
N_EDGES = 160000
HIDDEN = 256
N_MOLS = 200

NC = 2
NS = 16
OWN = 5120
NPAD = NC * OWN
EPT = N_EDGES // NS
CHUNK = 128
NFULL = EPT // CHUNK
TAIL = EPT - NFULL * CHUNK
OWN_T = NPAD // (NC * NS)
ACC_T = OWN_T + 16
EBLK = 1280

M_ACC = 384
M_TRASH = 200

_MESH = plsc.VectorSubcoreMesh(core_axis_name="c", subcore_axis_name="s")

_f32 = jnp.float32
_i32 = jnp.int32


def _sc_deg_body(dst_hbm, out_hbm, dbuf, sbuf, ones_v, z1, acc, sem):
    c = lax.axis_index("c")
    t = lax.axis_index("s")
    lo = c * OWN
    hi = OWN + c * (N_NODES - OWN)
    iota16 = lax.iota(_i32, 16)

    one16 = jnp.ones((16,), _f32)
    zero16 = jnp.zeros((16,), _f32)
    for j in range(CHUNK // 16):
        ones_v[pl.ds(16 * j, 16)] = one16
    for j in range(336 // 16):
        z1[pl.ds(16 * j, 16)] = zero16
    pltpu.sync_copy(z1, acc.at[pl.ds(t * 336, 336)])
    plsc.subcore_barrier()

    def flush(f, carry):
        eb = t * EPT + f * CHUNK
        pltpu.sync_copy(dst_hbm.at[pl.ds(eb, CHUNK)], dbuf)
        for j in range(CHUNK // 16):
            dv = dbuf[pl.ds(16 * j, 16)]
            owned = (dv >= lo) & (dv < hi)
            sbuf[pl.ds(16 * j, 16)] = jnp.where(owned, dv - lo,
                                                OWN + 16 * j + iota16)
        pltpu.sync_copy(ones_v, acc.at[sbuf], add=True)
        return carry

    lax.fori_loop(0, NFULL, flush, 0)

    eb = t * EPT + NFULL * CHUNK
    pltpu.sync_copy(dst_hbm.at[pl.ds(eb, TAIL)], dbuf.at[pl.ds(0, TAIL)])
    dv = dbuf[pl.ds(0, 16)]
    owned = (dv >= lo) & (dv < hi)
    sbuf[pl.ds(0, 16)] = jnp.where(owned, dv - lo, OWN + iota16)
    for j in range(1, CHUNK // 16):
        sbuf[pl.ds(16 * j, 16)] = OWN + 16 * j + iota16
    pltpu.sync_copy(ones_v, acc.at[sbuf], add=True)

    plsc.subcore_barrier()

    @pl.when(t < 10)
    def _():
        pltpu.sync_copy(acc.at[pl.ds(t * 512, 512)],
                        out_hbm.at[pl.ds(c * OWN + t * 512, 512)])


_sc_deg = functools.partial(
    pl.kernel,
    out_type=jax.ShapeDtypeStruct((NPAD,), _f32),
    mesh=_MESH,
    scratch_types=[
        pltpu.VMEM((CHUNK,), _i32),
        pltpu.VMEM((CHUNK,), _i32),
        pltpu.VMEM((CHUNK,), _f32),
        pltpu.VMEM((336,), _f32),
        pltpu.VMEM_SHARED((NS * 336,), _f32),
        pltpu.SemaphoreType.DMA,
    ],
)(_sc_deg_body)


def _sc_agg_body(g_hbm, src_hbm, dst_hbm, zer_hbm, out_hbm,
                 srcbuf, dstbuf, lst, gbuf, abuf, pkblk, csblk, fgblk,
                 staging, acc, sem):
    c = lax.axis_index("c")
    t = lax.axis_index("s")
    w = c * NS + t
    lo = w * OWN_T
    iota16 = lax.iota(_i32, 16)
    one16 = iota16 * 0 + 1
    _gdn = lax.GatherDimensionNumbers(offset_dims=(), collapsed_slice_dims=(0,),
                                      start_index_map=(0,))

    def _perm(v, idx):
        return lax.gather(v, idx.reshape(16, 1), _gdn, (1,),
                          mode=lax.GatherScatterMode.PROMISE_IN_BOUNDS)

    sh = [jnp.maximum(iota16 - k, 0) for k in (1, 2, 4, 8)]
    gm = [one16 - lax.shift_right_logical(iota16 - k, 31) for k in (1, 2, 4, 8)]
    NG = EBLK // 16

    pltpu.sync_copy(zer_hbm, acc)

    def rmw_flush(f):
        for g in range(CHUNK // 16):
            v2 = lst[pl.ds(f * CHUNK + 16 * g, 16)]
            gbuf[pl.ds(16 * g, 16)] = lax.bitwise_and(v2, 16383)
            abuf[pl.ds(16 * g, 16)] = lax.bitwise_and(lax.shift_right_logical(v2, 14), 1023)
        pltpu.async_copy(g_hbm.at[gbuf], staging, sem).wait()

        def grp(g, carry):
            dv16 = abuf[pl.ds(g * 16, 16)]
            for k in range(16):
                dloc2 = dv16[k]
                e = g * 16 + k
                for j in range(HIDDEN // 16):
                    acc[dloc2, pl.ds(16 * j, 16)] = (
                        acc[dloc2, pl.ds(16 * j, 16)]
                        + staging[e, pl.ds(16 * j, 16)])
            return carry

        lax.fori_loop(0, CHUNK // 16, grp, 0)

    def blk(b, cnt):
        pltpu.sync_copy(src_hbm.at[pl.ds(b * EBLK, EBLK)], srcbuf)
        pltpu.sync_copy(dst_hbm.at[pl.ds(b * EBLK, EBLK)], dstbuf)

        def la(i, carry):
            dv = dstbuf[pl.ds(i * 16, 16)]
            sv = srcbuf[pl.ds(i * 16, 16)]
            dvl = dv - lo
            m = one16 - lax.shift_right_logical(
                lax.bitwise_or(dvl, (OWN_T - 1) - dvl), 31)
            mf = iota16 * 0 - m
            dloc = lax.bitwise_or(
                lax.bitwise_and(dvl, mf),
                lax.bitwise_and(OWN_T + iota16, lax.bitwise_xor(mf, -1)))
            pkblk[pl.ds(i * 16, 16)] = (lax.shift_left(m, 24)
                                        + lax.shift_left(dloc, 14) + sv)
            return carry

        lax.fori_loop(0, NG, la, 0)

        def lb(i, cnt):
            cs = lax.shift_right_logical(pkblk[pl.ds(i * 16, 16)], 24)
            for si, g01 in zip(sh, gm):
                cs = cs + _perm(cs, si) * g01
            csblk[pl.ds(i * 16, 16)] = cs
            return cnt + cs[15]

        cnt0 = cnt
        cnt = lax.fori_loop(0, NG, lb, cnt)

        def lc(i, carry):
            cs2 = csblk[pl.ds(i * 16, 16)]
            fg = iota16 * 0
            for step in (8, 4):
                cand = fg + step
                okm = one16 - lax.shift_right_logical(
                    iota16 - _perm(cs2, cand - one16), 31)
                fg = fg + step * okm
            fgblk[pl.ds(i * 16, 16)] = fg
            return carry

        lax.fori_loop(0, NG, lc, 0)

        def ld(i, cnt2):
            cs3 = csblk[pl.ds(i * 16, 16)]
            fg2 = fgblk[pl.ds(i * 16, 16)]
            for step in (2, 1):
                cand = fg2 + step
                okm = one16 - lax.shift_right_logical(
                    iota16 - _perm(cs3, cand - one16), 31)
                fg2 = fg2 + step * okm
            lst[pl.ds(cnt2, 16)] = _perm(pkblk[pl.ds(i * 16, 16)], fg2)
            return cnt2 + cs3[15]

        lax.fori_loop(0, NG, ld, cnt0)

        def fl(f2, cnt):
            @pl.when(cnt >= CHUNK)
            def _():
                rmw_flush(0)
                for gmv in range((EBLK + CHUNK) // 16):
                    v3 = lst[pl.ds(CHUNK + 16 * gmv, 16)]
                    lst[pl.ds(16 * gmv, 16)] = v3

            return jnp.where(cnt >= CHUNK, cnt - CHUNK, cnt)

        return lax.fori_loop(0, EBLK // CHUNK + 1, fl, cnt)

    cnt = lax.fori_loop(0, N_EDGES // EBLK, blk, jnp.zeros((), _i32))

    junk = lax.shift_left(OWN_T + iota16, 14) + iota16
    for k2 in range(CHUNK // 16):
        lst[pl.ds(cnt + 16 * k2, 16)] = junk + 16 * k2
    rmw_flush(0)

    pltpu.sync_copy(acc.at[pl.ds(0, OWN_T)],
                    out_hbm.at[pl.ds(w * OWN_T, OWN_T)])


_sc_agg = functools.partial(
    pl.kernel,
    out_type=jax.ShapeDtypeStruct((NPAD, HIDDEN), _f32),
    mesh=_MESH,
    scratch_types=[
        pltpu.VMEM((EBLK,), _i32),
        pltpu.VMEM((EBLK,), _i32),
        pltpu.VMEM((EBLK + 2 * CHUNK, ), _i32),
        pltpu.VMEM((CHUNK,), _i32),
        pltpu.VMEM((CHUNK,), _i32),
        pltpu.VMEM((EBLK,), _i32),
        pltpu.VMEM((EBLK,), _i32),
        pltpu.VMEM((EBLK,), _i32),
        pltpu.VMEM((CHUNK, HIDDEN), _f32),
        pltpu.VMEM((ACC_T, HIDDEN), _f32),
        pltpu.SemaphoreType.DMA,
    ],
)(_sc_agg_body)


def _sc_mid_body(a_hbm, seg_hbm, out_hbm, staging, segbuf, acc, sem):
    c = lax.axis_index("c")
    t = lax.axis_index("s")
    w = c * NS + t
    base = w * 320
    z16 = jnp.zeros((16,), _f32)

    def zrow(i, carry):
        for j in range(HIDDEN // 16):
            acc[i, pl.ds(16 * j, 16)] = z16
        return carry

    lax.fori_loop(0, N_MOLS, zrow, 0)

    def do_chunk(off, sz):
        pltpu.sync_copy(seg_hbm.at[pl.ds(base + off, sz)], segbuf.at[pl.ds(0, sz)])
        pltpu.sync_copy(a_hbm.at[pl.ds(base + off, sz)], staging.at[pl.ds(0, sz)])

        def grp(g, carry):
            sg16 = segbuf[pl.ds(g * 16, 16)]
            for k in range(16):
                dloc = sg16[k]
                e = g * 16 + k
                for j in range(HIDDEN // 16):
                    acc[dloc, pl.ds(16 * j, 16)] = (
                        acc[dloc, pl.ds(16 * j, 16)]
                        + staging[e, pl.ds(16 * j, 16)])
            return carry

        lax.fori_loop(0, sz // 16, grp, 0)

    @pl.when(w < 31)
    def _():
        do_chunk(0, 128)
        do_chunk(128, 128)
        do_chunk(256, 64)

    @pl.when(w == 31)
    def _():
        do_chunk(0, 80)

    pltpu.sync_copy(acc, out_hbm.at[w])


_sc_mid = functools.partial(
    pl.kernel,
    out_type=jax.ShapeDtypeStruct((NC * NS, N_MOLS, HIDDEN), _f32),
    mesh=_MESH,
    scratch_types=[
        pltpu.VMEM((CHUNK, HIDDEN), _f32),
        pltpu.VMEM((CHUNK,), _i32),
        pltpu.VMEM((N_MOLS, HIDDEN), _f32),
        pltpu.SemaphoreType.DMA,
    ],
)(_sc_mid_body)


ROWS_BLK = 2048
GRID = NPAD // ROWS_BLK


def _lorentz_rescale(y, es):
    time = jax.nn.sigmoid(y[:, :1]) * es + 1.1
    xn = y[:, 1:]
    scale = (time * time - 1.0) / jnp.clip(
        jnp.sum(xn * xn, axis=-1, keepdims=True), 1e-8, None)
    return jnp.concatenate([time, xn * jnp.sqrt(scale)], axis=-1)


def _tc_first_body(x_ref, cnt_ref, w_ref, b_ref, s_ref, o_ref):
    x = x_ref[...]
    nrm = jnp.sqrt(jnp.clip(jnp.sum(x * x, axis=-1, keepdims=True), 1e-8, None))
    ep = jnp.exp(nrm)
    en = jnp.exp(-nrm)
    h = jnp.concatenate([0.5 * (ep + en), (0.5 * (ep - en)) / nrm * x], axis=-1)
    y = lax.dot_general(h, w_ref[...], (((1,), (0,)), ((), ())),
                        preferred_element_type=_f32) + b_ref[...]
    h = _lorentz_rescale(y, jnp.exp(s_ref[0, 0]))
    o_ref[...] = h * lax.rsqrt(cnt_ref[...] + 1.0)


def _tc_mid_body(S_ref, g_ref, cnt_ref, w_ref, b_ref, s_ref, o_ref):
    v = S_ref[...] + g_ref[...]
    v0 = v[:, :1]
    inner = jnp.sum(v * v, axis=-1, keepdims=True) - 2.0 * v0 * v0
    a = v * lax.rsqrt(jnp.clip(jnp.abs(inner), 1e-8, None))
    a = jnp.maximum(a, 0.0)
    y = lax.dot_general(a, w_ref[...], (((1,), (0,)), ((), ())),
                        preferred_element_type=_f32) + b_ref[...]
    h = _lorentz_rescale(y, jnp.exp(s_ref[0, 0]))
    o_ref[...] = h * lax.rsqrt(cnt_ref[...] + 1.0)


def _tc_norm_body(S_ref, g_ref, o_ref):
    v = S_ref[...] + g_ref[...]
    v0 = v[:, :1]
    inner = jnp.sum(v * v, axis=-1, keepdims=True) - 2.0 * v0 * v0
    o_ref[...] = v * lax.rsqrt(jnp.clip(jnp.abs(inner), 1e-8, None))


def _tc_final_body(p_ref, o_ref):
    v = p_ref[0]
    for k in range(1, NC * NS):
        v = v + p_ref[k]
    v0 = v[:, :1]
    inner = jnp.sum(v * v, axis=-1, keepdims=True) - 2.0 * v0 * v0
    o_ref[...] = v * lax.rsqrt(jnp.clip(jnp.abs(inner), 1e-8, None))


def _rows_spec(width):
    return pl.BlockSpec((ROWS_BLK, width), lambda i: (i, 0))


def _full_spec(shape):
    return pl.BlockSpec(shape, lambda i: tuple(0 for _ in shape))


def _tc_first(x, cnt2d, W, b2d, s2d):
    return pl.pallas_call(
        _tc_first_body,
        grid=(GRID,),
        in_specs=[_rows_spec(x.shape[1]), _rows_spec(1),
                  _full_spec(W.shape), _full_spec(b2d.shape), _full_spec(s2d.shape)],
        out_specs=_rows_spec(HIDDEN),
        out_shape=jax.ShapeDtypeStruct((NPAD, HIDDEN), _f32),
    )(x, cnt2d, W, b2d, s2d)


def _tc_mid(S, g, cnt2d, W, b2d, s2d):
    return pl.pallas_call(
        _tc_mid_body,
        grid=(GRID,),
        in_specs=[_rows_spec(HIDDEN), _rows_spec(HIDDEN), _rows_spec(1),
                  _full_spec(W.shape), _full_spec(b2d.shape), _full_spec(s2d.shape)],
        out_specs=_rows_spec(HIDDEN),
        out_shape=jax.ShapeDtypeStruct((NPAD, HIDDEN), _f32),
    )(S, g, cnt2d, W, b2d, s2d)


def _tc_norm(S, g):
    return pl.pallas_call(
        _tc_norm_body,
        grid=(GRID,),
        in_specs=[_rows_spec(HIDDEN), _rows_spec(HIDDEN)],
        out_specs=_rows_spec(HIDDEN),
        out_shape=jax.ShapeDtypeStruct((NPAD, HIDDEN), _f32),
    )(S, g)


def _tc_final(p):
    return pl.pallas_call(
        _tc_final_body,
        in_specs=[pl.BlockSpec((NC * NS, N_MOLS, HIDDEN), lambda: (0, 0, 0))],
        out_specs=pl.BlockSpec((N_MOLS, HIDDEN), lambda: (0, 0)),
        out_shape=jax.ShapeDtypeStruct((N_MOLS, HIDDEN), _f32),
    )(p)


def kernel(x, edge_index, segment_ids, W0, b0, s0, W1, b1, s1, W2, b2, s2,
           W3, b3, s3):
    src = edge_index[0]
    dst = edge_index[1]
    counts = _sc_deg(dst)
    cnt2d = counts.reshape(NPAD, 1)
    xp = jnp.pad(x, ((0, NPAD - N_NODES), (0, 0)))

    def prep(b, s):
        return b.reshape(1, HIDDEN), s.reshape(1, 1)

    zer = jnp.zeros((ACC_T, HIDDEN), _f32)
    b0r, s0r = prep(b0, s0)
    g = _tc_first(xp, cnt2d, W0, b0r, s0r)
    for W, b, s in ((W1, b1, s1), (W2, b2, s2), (W3, b3, s3)):
        S = _sc_agg(g, src, dst, zer)
        br, sr = prep(b, s)
        g = _tc_mid(S, g, cnt2d, W, br, sr)
    S = _sc_agg(g, src, dst, zer)
    a = _tc_norm(S, g)
    partials = _sc_mid(a, segment_ids)
    return _tc_final(partials)
